# baseline (speedup 1.0000x reference)
import jax
import jax.numpy as jnp
from jax.experimental import pallas as pl
from jax.experimental.pallas import tpu as pltpu

_BS, _L, _D = 128, 12, 8192


def _body(idx_ref, drop_ref, in_ref, out_ref):
    l0 = pl.program_id(0) * 3
    out_ref[...] = in_ref[...]

    def patch(b, _):
        dropped = drop_ref[b] > (1.0 - 0.2)
        for k in range(3):

            @pl.when(jnp.logical_and(dropped, idx_ref[b] == l0 + k))
            def _():
                out_ref[k, pl.ds(b, 1), :] = jnp.zeros((1, _D), jnp.float32)

        return 0

    jax.lax.fori_loop(0, _BS, patch, 0)


def _transposed_call(index, drop_rand, x_t):
    grid_spec = pltpu.PrefetchScalarGridSpec(
        num_scalar_prefetch=2,
        grid=(_L // 3,),
        in_specs=[
            pl.BlockSpec((3, _BS, _D), lambda l, idx_ref, drop_ref: (l, 0, 0)),
        ],
        out_specs=pl.BlockSpec((3, _BS, _D), lambda l, idx_ref, drop_ref: (l, 0, 0)),
    )
    return pl.pallas_call(
        _body,
        grid_spec=grid_spec,
        out_shape=jax.ShapeDtypeStruct((_L, _BS, _D), jnp.float32),
        compiler_params=pltpu.CompilerParams(
            dimension_semantics=("arbitrary",),
        ),
    )(index, drop_rand, x_t)


@jax.jit
def kernel(inputs, index, drop_rand):
    x_t = jnp.transpose(inputs, (1, 0, 2))
    out_t = _transposed_call(index, drop_rand, x_t)
    return jnp.transpose(out_t, (1, 0, 2))


# speedup vs baseline: 1.0104x; 1.0104x over previous
import jax
import jax.numpy as jnp
from jax.experimental import pallas as pl
from jax.experimental.pallas import tpu as pltpu

_BS, _L, _D = 128, 12, 8192


def _body(idx_ref, drop_ref, in_ref, out_ref):
    l0 = pl.program_id(0) * 2
    out_ref[...] = in_ref[...]

    def patch(b, _):
        dropped = drop_ref[b] > (1.0 - 0.2)
        for k in range(2):

            @pl.when(jnp.logical_and(dropped, idx_ref[b] == l0 + k))
            def _():
                out_ref[k, pl.ds(b, 1), :] = jnp.zeros((1, _D), jnp.float32)

        return 0

    jax.lax.fori_loop(0, _BS, patch, 0)


def _transposed_call(index, drop_rand, x_t):
    grid_spec = pltpu.PrefetchScalarGridSpec(
        num_scalar_prefetch=2,
        grid=(_L // 2,),
        in_specs=[
            pl.BlockSpec((2, _BS, _D), lambda l, idx_ref, drop_ref: (l, 0, 0)),
        ],
        out_specs=pl.BlockSpec((2, _BS, _D), lambda l, idx_ref, drop_ref: (l, 0, 0)),
    )
    return pl.pallas_call(
        _body,
        grid_spec=grid_spec,
        out_shape=jax.ShapeDtypeStruct((_L, _BS, _D), jnp.float32),
        compiler_params=pltpu.CompilerParams(
            dimension_semantics=("arbitrary",),
        ),
    )(index, drop_rand, x_t)


@jax.jit
def kernel(inputs, index, drop_rand):
    x_t = jnp.transpose(inputs, (1, 0, 2))
    out_t = _transposed_call(index, drop_rand, x_t)
    return jnp.transpose(out_t, (1, 0, 2))
